# baseline (speedup 1.0000x reference)
import functools

import jax
import jax.numpy as jnp
from jax import lax
from jax.experimental import pallas as pl
from jax.experimental.pallas import tpu as pltpu
from jax.experimental.pallas import tpu_sc as plsc
B = 2
S = 4096
D_MODEL = 1024
HEAD_DIM = 128
HALF = HEAD_DIM // 2

NC = 2
NS = 16
NW = NC * NS
ROWS = B * S
RPW = ROWS // NW
CHUNK = 32
NCHUNK = RPW // CHUNK

MASK_BS = 256
NBUF = 4
NBLK = S // MASK_BS
NEG = float(jnp.finfo(jnp.float32).min)

_sc_mesh = plsc.VectorSubcoreMesh(core_axis_name="c", subcore_axis_name="s")


WPB = S // RPW


@functools.partial(
    pl.kernel,
    mesh=_sc_mesh,
    out_type=jax.ShapeDtypeStruct((B, S, D_MODEL), jnp.float32),
    scratch_types=[
        pltpu.VMEM((CHUNK,), jnp.int32),
        pltpu.VMEM((CHUNK, D_MODEL), jnp.float32),
        pltpu.SemaphoreType.DMA,
    ],
)
def _sc_gather(ids_hbm, table_hbm, out_hbm, idx_v, rows_v, sem):
    wid = lax.axis_index("s") * NC + lax.axis_index("c")
    b = wid // WPB
    r0 = (wid % WPB) * RPW
    for c in range(NCHUNK):
        off = r0 + c * CHUNK
        pltpu.sync_copy(ids_hbm.at[b, pl.ds(off, CHUNK)], idx_v)
        pltpu.async_copy(table_hbm.at[idx_v], rows_v, sem).wait()
        pltpu.sync_copy(rows_v, out_hbm.at[b, pl.ds(off, CHUNK), :])


def _mask_body(am_ref, invf_ref, out_ref, cos_ref, sin_ref, pos_ref,
               cache_ref, bufs, sems):
    b = pl.program_id(0)
    i = pl.program_id(1)
    k = b * NBLK + i
    buf = k % NBUF

    def _dma(j, bb, ii):
        return pltpu.make_async_copy(
            bufs.at[j],
            out_ref.at[bb, pl.ds(ii * MASK_BS, MASK_BS), :],
            sems.at[j],
        )

    @pl.when(k >= NBUF)
    def _():
        _dma(buf, b, i).wait()

    rows = lax.broadcasted_iota(jnp.int32, (MASK_BS, S), 0) + i * MASK_BS
    cols = lax.broadcasted_iota(jnp.int32, (MASK_BS, S), 1)
    pad_row = jnp.where(b == 0, am_ref[0, :], am_ref[1, :])
    pad = (pad_row == 0)[None, :]
    m = (cols > rows) | pad
    bufs[buf] = jnp.where(m, NEG, 0.0).astype(jnp.float32)
    _dma(buf, b, i).start()

    @pl.when(k == 0)
    def _():
        NQ = S // 64
        w_half = invf_ref[0, :][None, :]
        w = jnp.concatenate([w_half, w_half], axis=-1)
        q_ang = (lax.broadcasted_iota(jnp.int32, (NQ, HEAD_DIM), 0) * 64
                 ).astype(jnp.float32) * w
        r_ang = lax.broadcasted_iota(jnp.int32, (64, HEAD_DIM), 0
                                     ).astype(jnp.float32) * w
        cq = jnp.cos(q_ang)[:, None, :]
        sq = jnp.sin(q_ang)[:, None, :]
        cr = jnp.cos(r_ang)[None, :, :]
        sr = jnp.sin(r_ang)[None, :, :]
        cos_ref[0] = (cq * cr - sq * sr).reshape(S, HEAD_DIM)
        sin_ref[0] = (sq * cr + cq * sr).reshape(S, HEAD_DIM)
        ids = lax.broadcasted_iota(jnp.int32, (1, S), 1)
        pos_ref[...] = ids
        cache_ref[...] = ids

    @pl.when(k == B * NBLK - 1)
    def _():
        for j in range(NBUF):
            _dma(j, b, i).wait()


def kernel(input_ids, attention_mask, embed_table, inv_freq):
    hidden = _sc_gather(input_ids, embed_table)

    mask3, cos, sin, pos, cache = pl.pallas_call(
        _mask_body,
        grid=(B, NBLK),
        in_specs=[
            pl.BlockSpec((B, S), lambda b, i: (0, 0)),
            pl.BlockSpec((1, HALF), lambda b, i: (0, 0)),
        ],
        out_specs=[
            pl.BlockSpec(memory_space=pl.ANY),
            pl.BlockSpec((1, S, HEAD_DIM), lambda b, i: (0, 0, 0)),
            pl.BlockSpec((1, S, HEAD_DIM), lambda b, i: (0, 0, 0)),
            pl.BlockSpec((1, S), lambda b, i: (0, 0)),
            pl.BlockSpec((1, S), lambda b, i: (0, 0)),
        ],
        out_shape=[
            jax.ShapeDtypeStruct((B, S, S), jnp.float32),
            jax.ShapeDtypeStruct((1, S, HEAD_DIM), jnp.float32),
            jax.ShapeDtypeStruct((1, S, HEAD_DIM), jnp.float32),
            jax.ShapeDtypeStruct((1, S), jnp.int32),
            jax.ShapeDtypeStruct((1, S), jnp.int32),
        ],
        scratch_shapes=[
            pltpu.VMEM((NBUF, MASK_BS, S), jnp.float32),
            pltpu.SemaphoreType.DMA((NBUF,)),
        ],
    )(attention_mask, inv_freq.reshape(1, HALF))
    causal_mask = mask3[:, None, :, :]

    return (hidden, causal_mask, pos, cache.reshape(S), cos, sin)


# speedup vs baseline: 1.0095x; 1.0095x over previous
import functools

import jax
import jax.numpy as jnp
from jax import lax
from jax.experimental import pallas as pl
from jax.experimental.pallas import tpu as pltpu
from jax.experimental.pallas import tpu_sc as plsc
B = 2
S = 4096
D_MODEL = 1024
HEAD_DIM = 128
HALF = HEAD_DIM // 2

NC = 2
NS = 16
NW = NC * NS
ROWS = B * S
RPW = ROWS // NW
CHUNK = 32
NCHUNK = RPW // CHUNK
GNB = 3

MASK_BS = 256
NBUF = 4
NBLK = S // MASK_BS
NEG = float(jnp.finfo(jnp.float32).min)

_sc_mesh = plsc.VectorSubcoreMesh(core_axis_name="c", subcore_axis_name="s")


WPB = S // RPW


@functools.partial(
    pl.kernel,
    mesh=_sc_mesh,
    out_type=jax.ShapeDtypeStruct((B, S, D_MODEL), jnp.float32),
    scratch_types=[
        pltpu.VMEM((GNB, CHUNK), jnp.int32),
        pltpu.VMEM((GNB, CHUNK, D_MODEL), jnp.float32),
        pltpu.SemaphoreType.DMA((GNB,)),
        pltpu.SemaphoreType.DMA((GNB,)),
    ],
)
def _sc_gather(ids_hbm, table_hbm, out_hbm, idx_v, rows_v, gsem, wsem):
    wid = lax.axis_index("s") * NC + lax.axis_index("c")
    b = wid // WPB
    r0 = (wid % WPB) * RPW

    def _wb(c):
        s = c % GNB
        off = r0 + c * CHUNK
        return pltpu.make_async_copy(
            rows_v.at[s], out_hbm.at[b, pl.ds(off, CHUNK), :], wsem.at[s])

    def _gwait(c):
        s = c % GNB
        pltpu.make_async_copy(
            table_hbm.at[idx_v.at[s]], rows_v.at[s], gsem.at[s]).wait()

    for c in range(NCHUNK):
        s = c % GNB
        if c >= GNB:
            _wb(c - GNB).wait()
        pltpu.sync_copy(ids_hbm.at[b, pl.ds(r0 + c * CHUNK, CHUNK)], idx_v.at[s])
        pltpu.async_copy(table_hbm.at[idx_v.at[s]], rows_v.at[s], gsem.at[s])
        if c >= 1:
            _gwait(c - 1)
            _wb(c - 1).start()
    _gwait(NCHUNK - 1)
    _wb(NCHUNK - 1).start()
    for c in range(NCHUNK - GNB, NCHUNK):
        _wb(c).wait()


def _mask_body(am_ref, invf_ref, out_ref, cos_ref, sin_ref, pos_ref,
               cache_ref, bufs, sems):
    b = pl.program_id(0)
    i = pl.program_id(1)
    k = b * NBLK + i
    buf = k % NBUF

    def _dma(j, bb, ii):
        return pltpu.make_async_copy(
            bufs.at[j],
            out_ref.at[bb, pl.ds(ii * MASK_BS, MASK_BS), :],
            sems.at[j],
        )

    @pl.when(k >= NBUF)
    def _():
        _dma(buf, b, i).wait()

    rows = lax.broadcasted_iota(jnp.int32, (MASK_BS, S), 0) + i * MASK_BS
    cols = lax.broadcasted_iota(jnp.int32, (MASK_BS, S), 1)
    pad_row = jnp.where(b == 0, am_ref[0, :], am_ref[1, :])
    pad = (pad_row == 0)[None, :]
    m = (cols > rows) | pad
    bufs[buf] = jnp.where(m, NEG, 0.0).astype(jnp.float32)
    _dma(buf, b, i).start()

    @pl.when(k == 0)
    def _():
        NQ = S // 64
        w_half = invf_ref[0, :][None, :]
        w = jnp.concatenate([w_half, w_half], axis=-1)
        q_ang = (lax.broadcasted_iota(jnp.int32, (NQ, HEAD_DIM), 0) * 64
                 ).astype(jnp.float32) * w
        r_ang = lax.broadcasted_iota(jnp.int32, (64, HEAD_DIM), 0
                                     ).astype(jnp.float32) * w
        cq = jnp.cos(q_ang)[:, None, :]
        sq = jnp.sin(q_ang)[:, None, :]
        cr = jnp.cos(r_ang)[None, :, :]
        sr = jnp.sin(r_ang)[None, :, :]
        cos_ref[0] = (cq * cr - sq * sr).reshape(S, HEAD_DIM)
        sin_ref[0] = (sq * cr + cq * sr).reshape(S, HEAD_DIM)
        ids = lax.broadcasted_iota(jnp.int32, (1, S), 1)
        pos_ref[...] = ids
        cache_ref[...] = ids

    @pl.when(k == B * NBLK - 1)
    def _():
        for j in range(NBUF):
            _dma(j, b, i).wait()


def kernel(input_ids, attention_mask, embed_table, inv_freq):
    hidden = _sc_gather(input_ids, embed_table)

    mask3, cos, sin, pos, cache = pl.pallas_call(
        _mask_body,
        grid=(B, NBLK),
        in_specs=[
            pl.BlockSpec((B, S), lambda b, i: (0, 0)),
            pl.BlockSpec((1, HALF), lambda b, i: (0, 0)),
        ],
        out_specs=[
            pl.BlockSpec(memory_space=pl.ANY),
            pl.BlockSpec((1, S, HEAD_DIM), lambda b, i: (0, 0, 0)),
            pl.BlockSpec((1, S, HEAD_DIM), lambda b, i: (0, 0, 0)),
            pl.BlockSpec((1, S), lambda b, i: (0, 0)),
            pl.BlockSpec((1, S), lambda b, i: (0, 0)),
        ],
        out_shape=[
            jax.ShapeDtypeStruct((B, S, S), jnp.float32),
            jax.ShapeDtypeStruct((1, S, HEAD_DIM), jnp.float32),
            jax.ShapeDtypeStruct((1, S, HEAD_DIM), jnp.float32),
            jax.ShapeDtypeStruct((1, S), jnp.int32),
            jax.ShapeDtypeStruct((1, S), jnp.int32),
        ],
        scratch_shapes=[
            pltpu.VMEM((NBUF, MASK_BS, S), jnp.float32),
            pltpu.SemaphoreType.DMA((NBUF,)),
        ],
    )(attention_mask, inv_freq.reshape(1, HALF))
    causal_mask = mask3[:, None, :, :]

    return (hidden, causal_mask, pos, cache.reshape(S), cos, sin)
